# Initial kernel scaffold; baseline (speedup 1.0000x reference)
import jax
import jax.numpy as jnp
from jax.experimental import pallas as pl


def kernel(state, item_embeddings, M):
    raise NotImplementedError("write your pallas kernel here")


# speedup vs baseline: 16.1639x; 16.1639x over previous
import functools

import jax
import jax.numpy as jnp
from jax import lax
from jax.experimental import pallas as pl
from jax.experimental.pallas import tpu as pltpu
from jax.experimental.pallas import tpu_sc as plsc

B = 1024
D = 64
V = 100000
VPAD = 100352
W = 1024
NT = VPAD // W
G = VPAD // 128
GP = 896
NPAD = VPAD - V
K_GRP = 128
K_CAND = 512
TOPK = 100
NC, NS = 2, 16
NW = NC * NS
ROWS_PER_W = B // NW


def _mm_body(s_ref, e_ref, lg_ref, gm_ref, m_ref, sm_ref, mscr, sscr):
    i = pl.program_id(0)

    @pl.when(i == 0)
    def _():
        mscr[...] = jnp.full((B, 1), -jnp.inf, jnp.float32)
        sscr[...] = jnp.zeros((B, 1), jnp.float32)

    x = lax.dot_general(s_ref[...], e_ref[...], (((1,), (1,)), ((), ())),
                        preferred_element_type=jnp.float32)
    lg_ref[...] = x
    gm_ref[...] = jnp.max(x.reshape(B, W // 128, 128), axis=2).reshape(
        1, B, W // 128)

    tm = jnp.max(x, axis=1, keepdims=True)
    nm = jnp.maximum(mscr[...], tm)
    sscr[...] = (sscr[...] * jnp.exp(mscr[...] - nm)
                 + jnp.sum(jnp.exp(x - nm), axis=1, keepdims=True))
    mscr[...] = nm

    @pl.when(i == NT - 1)
    def _():
        sscr[...] = sscr[...] - NPAD * jnp.exp(-mscr[...])

    m_ref[...] = mscr[...]
    sm_ref[...] = sscr[...]


def _tau_body(g_ref, t_ref):
    g = g_ref[...]
    ku = lax.bitcast_convert_type(g, jnp.uint32)
    sign = (ku >> jnp.uint32(31)).astype(jnp.int32)
    key = jnp.where(sign == 1, ~ku, ku | jnp.uint32(0x80000000))
    k18 = (key >> jnp.uint32(14)).astype(jnp.int32)
    t = jnp.zeros((B, 1), jnp.int32)
    for bit in range(17, -1, -1):
        cand = t + (1 << bit)
        cnt = jnp.sum((k18 >= cand).astype(jnp.float32), axis=1, keepdims=True)
        t = jnp.where(cnt >= float(TOPK), cand, t)
    kt = t.astype(jnp.uint32) << jnp.uint32(14)
    s2 = (kt >> jnp.uint32(31)).astype(jnp.int32)
    u = jnp.where(s2 == 1, kt ^ jnp.uint32(0x80000000), ~kt)
    tau = lax.bitcast_convert_type(u, jnp.float32)
    t_ref[...] = jnp.broadcast_to(tau, (B, 128))


def _sc_body(lg_hbm, gm_hbm, tau_hbm, cv_hbm, ci_hbm,
             gmax_v, tau_v, gid_v, grp_v, cv_v, ci_v, sem):
    wid = lax.axis_index("s") * NC + lax.axis_index("c")
    iota16 = lax.iota(jnp.int32, 16)
    one = jnp.ones((16,), jnp.int32)
    zero = jnp.zeros((16,), jnp.int32)

    def row_body(k, carry):
        r = wid * ROWS_PER_W + k
        base = r * G
        pltpu.sync_copy(gm_hbm.at[r], gmax_v)
        pltpu.sync_copy(tau_hbm.at[r], tau_v)
        tau = tau_v[pl.ds(0, 16)]
        for ii in range(K_CAND // 16):
            cv_v[pl.ds(ii * 16, 16)] = jnp.full((16,), -jnp.inf, jnp.float32)
            ci_v[pl.ds(ii * 16, 16)] = zero
        for ii in range(K_GRP // 16):
            gid_v[pl.ds(ii * 16, 16)] = zero + base

        cnt = zero
        for j in range(G // 16):
            v = gmax_v[pl.ds(j * 16, 16)]
            m = v >= tau
            pos = plsc.cumsum(jnp.where(m, one, zero))
            tgt = jnp.minimum(jnp.maximum(cnt + pos - 1, 0), K_GRP - 1)
            plsc.store_scatter(gid_v, [tgt], base + j * 16 + iota16, mask=m)
            cnt = cnt + plsc.all_reduce_population_count(m)
        ngrp = cnt

        pltpu.async_copy(lg_hbm.at[gid_v], grp_v, sem).wait()

        def grp_body(s, cnt2):
            svec = zero + s
            gvalid = svec < ngrp
            gid = plsc.load_gather(gid_v, [svec])
            gl = (gid - base) * 128
            c = cnt2
            for j in range(8):
                ev = plsc.load_gather(grp_v, [svec, j * 16 + iota16])
                eidx = gl + (j * 16) + iota16
                m2 = (ev >= tau) & gvalid & (eidx < V)
                pos2 = plsc.cumsum(jnp.where(m2, one, zero))
                t2 = jnp.minimum(jnp.maximum(c + pos2 - 1, 0), K_CAND - 1)
                plsc.store_scatter(cv_v, [t2], ev, mask=m2)
                plsc.store_scatter(ci_v, [t2], eidx, mask=m2)
                c = c + plsc.all_reduce_population_count(m2)
            return c

        lax.fori_loop(0, K_GRP, grp_body, zero)
        pltpu.sync_copy(cv_v, cv_hbm.at[r])
        pltpu.sync_copy(ci_v, ci_hbm.at[r])
        return carry

    lax.fori_loop(0, ROWS_PER_W, row_body, jnp.int32(0))


def _sel_body(cv_ref, ci_ref, m_ref, s_ref, oi_ref, op_ref):
    v = cv_ref[...]
    ix = ci_ref[...]
    lane = lax.broadcasted_iota(jnp.int32, (B, 128), 1)
    acc_p = jnp.full((B, 128), -jnp.inf, jnp.float32)
    acc_i = jnp.zeros((B, 128), jnp.int32)
    big = jnp.int32(2147483647)
    for k in range(TOPK):
        m = jnp.max(v, axis=1, keepdims=True)
        sel = v == m
        pick = jnp.min(jnp.where(sel, ix, big), axis=1, keepdims=True)
        v = jnp.where(ix == pick, -jnp.inf, v)
        acc_p = jnp.where(lane == k, m, acc_p)
        acc_i = jnp.where(lane == k, pick, acc_i)
    probs = jnp.exp(acc_p - m_ref[...]) / s_ref[...]
    oi_ref[...] = acc_i[:, :TOPK]
    op_ref[...] = probs[:, :TOPK]


def kernel(state, item_embeddings, M):
    f32 = jnp.float32
    e_pad = jnp.concatenate(
        [item_embeddings, jnp.zeros((VPAD - V, D), f32)], axis=0)

    logits, gmax, rmax, rsum = pl.pallas_call(
        _mm_body,
        grid=(NT,),
        in_specs=[
            pl.BlockSpec((B, D), lambda i: (0, 0)),
            pl.BlockSpec((W, D), lambda i: (i, 0)),
        ],
        out_specs=[
            pl.BlockSpec((B, W), lambda i: (0, i)),
            pl.BlockSpec((1, B, W // 128), lambda i: (i, 0, 0)),
            pl.BlockSpec((B, 1), lambda i: (0, 0)),
            pl.BlockSpec((B, 1), lambda i: (0, 0)),
        ],
        out_shape=[
            jax.ShapeDtypeStruct((B, VPAD), f32),
            jax.ShapeDtypeStruct((NT, B, W // 128), f32),
            jax.ShapeDtypeStruct((B, 1), f32),
            jax.ShapeDtypeStruct((B, 1), f32),
        ],
        scratch_shapes=[
            pltpu.VMEM((B, 1), f32),
            pltpu.VMEM((B, 1), f32),
        ],
        compiler_params=pltpu.CompilerParams(
            dimension_semantics=("arbitrary",)),
    )(state, e_pad)

    gmax = jnp.transpose(gmax, (1, 0, 2)).reshape(B, G)
    gmax_p = jnp.pad(gmax, ((0, 0), (0, GP - G)), constant_values=-jnp.inf)
    tau = pl.pallas_call(
        _tau_body,
        out_shape=jax.ShapeDtypeStruct((B, 128), f32),
    )(gmax_p)

    sc_fn = functools.partial(
        pl.kernel,
        mesh=plsc.VectorSubcoreMesh(core_axis_name="c", subcore_axis_name="s"),
        out_type=[
            jax.ShapeDtypeStruct((B, K_CAND), f32),
            jax.ShapeDtypeStruct((B, K_CAND), jnp.int32),
        ],
        scratch_types=[
            pltpu.VMEM((G,), f32),
            pltpu.VMEM((128,), f32),
            pltpu.VMEM((K_GRP,), jnp.int32),
            pltpu.VMEM((K_GRP, 128), f32),
            pltpu.VMEM((K_CAND,), f32),
            pltpu.VMEM((K_CAND,), jnp.int32),
            pltpu.SemaphoreType.DMA,
        ],
        compiler_params=pltpu.CompilerParams(needs_layout_passes=False),
    )(_sc_body)
    cand_v, cand_i = sc_fn(logits.reshape(B * G, 128), gmax, tau)

    items0, probs = pl.pallas_call(
        _sel_body,
        out_shape=[
            jax.ShapeDtypeStruct((B, TOPK), jnp.int32),
            jax.ShapeDtypeStruct((B, TOPK), f32),
        ],
    )(cand_v, cand_i, rmax, rsum)

    items = items0 + (jnp.asarray(M, jnp.int32) - TOPK)
    return items, probs
